# Initial kernel scaffold; baseline (speedup 1.0000x reference)
import jax
import jax.numpy as jnp
from jax.experimental import pallas as pl


def kernel(tensor, batch_idx, atomic_numbers, element_references):
    raise NotImplementedError("write your pallas kernel here")


# speedup vs baseline: 18.3080x; 18.3080x over previous
import functools
import jax
import jax.numpy as jnp
from jax import lax
from jax.experimental import pallas as pl
from jax.experimental.pallas import tpu as pltpu
from jax.experimental.pallas import tpu_sc as plsc

NC = 2
NS = 16
NW = NC * NS
LANES = 16


def _sc_partials(idx_ext, vals, num_segments, interpret=False):
    n = vals.shape[0]
    chunk = n // NW
    assert chunk * NW == n
    w = chunk
    for cand in (16384, 12800, 10000, 8192, 6250, 5120, 4096, 2048):
        if chunk % cand == 0 and cand % 16 == 0:
            w = cand
            break
    nwin = chunk // w
    nvec = w // LANES
    nacc = num_segments // LANES
    assert num_segments % LANES == 0

    mesh = plsc.VectorSubcoreMesh(
        core_axis_name="c", subcore_axis_name="s", num_cores=NC, num_subcores=NS
    )

    @functools.partial(
        pl.kernel,
        out_type=jax.ShapeDtypeStruct((NW, num_segments), jnp.float32),
        mesh=mesh,
        scratch_types=[
            pltpu.VMEM((w + LANES,), jnp.int32),
            pltpu.VMEM((w,), jnp.float32),
            pltpu.VMEM((num_segments,), jnp.float32),
        ],
        compiler_params=pltpu.CompilerParams(needs_layout_passes=False),
        interpret=interpret,
    )
    def sc_kernel(idx_hbm, val_hbm, part_hbm, idx_buf, val_buf, acc):
        cid = lax.axis_index("c")
        sid = lax.axis_index("s")
        wid = sid * NC + cid
        base = pl.multiple_of(wid * chunk, 8)

        def zero_body(i, _):
            acc[pl.ds(i * LANES, LANES)] = jnp.zeros((LANES,), jnp.float32)
            return 0

        lax.fori_loop(0, nacc, zero_body, 0)

        def win_body(widx, _):
            wb = pl.multiple_of(base + widx * w, 8)
            pltpu.sync_copy(idx_hbm.at[pl.ds(wb, w + LANES)], idx_buf)
            pltpu.sync_copy(val_hbm.at[pl.ds(wb, w)], val_buf)

            def vec_body(t, _):
                iv = idx_buf[pl.ds(t * LANES, LANES)]
                vv = val_buf[pl.ds(t * LANES, LANES)]
                plsc.addupdate_scatter(acc, [iv], vv)
                return 0

            lax.fori_loop(0, nvec, vec_body, 0)
            return 0

        lax.fori_loop(0, nwin, win_body, 0)
        pltpu.sync_copy(acc, part_hbm.at[wid])

    return sc_kernel(idx_ext, vals)


def _tc_combine(tensor2d, partials, interpret=False):
    def body(t_ref, p_ref, o_ref):
        o_ref[...] = t_ref[...] - jnp.sum(p_ref[...], axis=0, keepdims=True)

    return pl.pallas_call(
        body,
        out_shape=jax.ShapeDtypeStruct(tensor2d.shape, tensor2d.dtype),
        interpret=interpret,
    )(tensor2d, partials)


def kernel(tensor, batch_idx, atomic_numbers, element_references):
    num_segments = tensor.shape[0]
    idx = batch_idx.astype(jnp.int32)
    idx_ext = jnp.concatenate(
        [idx, jnp.full((LANES,), num_segments, dtype=jnp.int32)]
    )
    vals = atomic_numbers.astype(jnp.float32)
    partials = _sc_partials(idx_ext, vals, num_segments)
    out2d = _tc_combine(tensor.reshape(1, -1), partials)
    return out2d.reshape(tensor.shape)


# speedup vs baseline: 29.7058x; 1.6226x over previous
import functools
import jax
import jax.numpy as jnp
from jax import lax
from jax.experimental import pallas as pl
from jax.experimental.pallas import tpu as pltpu
from jax.experimental.pallas import tpu_sc as plsc

NC = 2
NS = 16
NW = NC * NS
LANES = 16


def _pick_window(chunk):
    for cand in (16384, 12800, 10000, 8192, 6250, 5120, 4096, 2048):
        if chunk % cand == 0 and cand % 16 == 0:
            return cand
    return chunk


def _sc_partials(idx_ext, vals, num_segments):
    n = vals.shape[0]
    chunk = n // NW
    assert chunk * NW == n
    w = _pick_window(chunk)
    nwin = chunk // w
    nvec = w // LANES
    acc_len = num_segments + LANES
    assert num_segments % LANES == 0

    mesh = plsc.VectorSubcoreMesh(
        core_axis_name="c", subcore_axis_name="s", num_cores=NC, num_subcores=NS
    )

    @functools.partial(
        pl.kernel,
        out_type=jax.ShapeDtypeStruct((NW, acc_len), jnp.float32),
        mesh=mesh,
        scratch_types=[
            pltpu.VMEM((w + LANES,), jnp.int32),
            pltpu.VMEM((w + LANES,), jnp.int32),
            pltpu.VMEM((w,), jnp.float32),
            pltpu.VMEM((w,), jnp.float32),
            pltpu.VMEM((acc_len,), jnp.float32),
            pltpu.SemaphoreType.DMA,
            pltpu.SemaphoreType.DMA,
            pltpu.SemaphoreType.DMA,
            pltpu.SemaphoreType.DMA,
        ],
        compiler_params=pltpu.CompilerParams(needs_layout_passes=False),
    )
    def sc_kernel(idx_hbm, val_hbm, part_hbm,
                  idx0, idx1, val0, val1, acc, si0, si1, sv0, sv1):
        cid = lax.axis_index("c")
        sid = lax.axis_index("s")
        wid = sid * NC + cid
        base = pl.multiple_of(wid * chunk, 8)

        def zero_body(i, _):
            acc[pl.ds(i * LANES, LANES)] = jnp.zeros((LANES,), jnp.float32)
            return 0

        lax.fori_loop(0, acc_len // LANES, zero_body, 0)

        idx_bufs = (idx0, idx1)
        val_bufs = (val0, val1)
        idx_sems = (si0, si1)
        val_sems = (sv0, sv1)

        def start(widx):
            p = widx % 2
            wb = pl.multiple_of(base + widx * w, 8)
            di = pltpu.async_copy(
                idx_hbm.at[pl.ds(wb, w + LANES)], idx_bufs[p], idx_sems[p]
            )
            dv = pltpu.async_copy(val_hbm.at[pl.ds(wb, w)], val_bufs[p], val_sems[p])
            return di, dv

        pending = start(0)
        carry0 = jnp.float32(0.0)

        for widx in range(nwin):
            p = widx % 2
            di, dv = pending
            di.wait()
            dv.wait()
            if widx + 1 < nwin:
                pending = start(widx + 1)
            if widx == nwin - 1:
                idx_bufs[p][pl.ds(w, LANES)] = jnp.full(
                    (LANES,), num_segments, dtype=jnp.int32
                )
            idxb = idx_bufs[p]
            valb = val_bufs[p]

            def vec_body(t, carry):
                iv = idxb[pl.ds(t * LANES, LANES)]
                nv = idxb[pl.ds(t * LANES + 1, LANES)]
                vv = valb[pl.ds(t * LANES, LANES)]
                g = plsc.cumsum(vv) + carry
                e = iv != nv
                plsc.addupdate_scatter(acc, [iv], g, mask=e)
                plsc.addupdate_scatter(acc, [nv], -g, mask=e)
                return carry + jnp.sum(vv)

            carry0 = lax.fori_loop(0, nvec, vec_body, carry0, unroll=5)

        pltpu.sync_copy(acc, part_hbm.at[wid])

    return sc_kernel(idx_ext, vals)


def _tc_combine(tensor2d, partials):
    n = tensor2d.shape[1]

    def body(t_ref, p_ref, o_ref):
        o_ref[...] = t_ref[...] - jnp.sum(p_ref[:, :n], axis=0, keepdims=True)

    return pl.pallas_call(
        body,
        out_shape=jax.ShapeDtypeStruct(tensor2d.shape, tensor2d.dtype),
    )(tensor2d, partials)


def kernel(tensor, batch_idx, atomic_numbers, element_references):
    num_segments = tensor.shape[0]
    idx = batch_idx.astype(jnp.int32)
    idx_ext = jnp.concatenate(
        [idx, jnp.full((LANES,), num_segments, dtype=jnp.int32)]
    )
    vals = atomic_numbers.astype(jnp.float32)
    partials = _sc_partials(idx_ext, vals, num_segments)
    out2d = _tc_combine(tensor.reshape(1, -1), partials)
    return out2d.reshape(tensor.shape)


# speedup vs baseline: 52.8259x; 1.7783x over previous
import functools
import jax
import jax.numpy as jnp
from jax import lax
from jax.experimental import pallas as pl
from jax.experimental.pallas import tpu as pltpu
from jax.experimental.pallas import tpu_sc as plsc

NC = 2
NS = 16
NW = NC * NS
LANES = 16


def _pick_window(chunk):
    for cand in (10000, 20000, 12800, 8192, 6250, 5000, 4096, 2000, 1040, 16):
        if chunk % cand == 0 and cand % LANES == 0:
            return cand
    return chunk


def _sc_partials(idx, vals, num_segments):
    n = vals.shape[0]
    chunk = n // NW
    assert chunk * NW == n
    w = _pick_window(chunk)
    nwin = chunk // w
    sub = w // LANES
    assert num_segments % LANES == 0

    mesh = plsc.VectorSubcoreMesh(
        core_axis_name="c", subcore_axis_name="s", num_cores=NC, num_subcores=NS
    )

    @functools.partial(
        pl.kernel,
        out_type=jax.ShapeDtypeStruct((NW, num_segments), jnp.float32),
        mesh=mesh,
        scratch_types=[
            pltpu.VMEM((w,), jnp.int32),
            pltpu.VMEM((w,), jnp.int32),
            pltpu.VMEM((w,), jnp.float32),
            pltpu.VMEM((w,), jnp.float32),
            pltpu.VMEM((num_segments,), jnp.float32),
            pltpu.SemaphoreType.DMA,
            pltpu.SemaphoreType.DMA,
            pltpu.SemaphoreType.DMA,
            pltpu.SemaphoreType.DMA,
        ],
        compiler_params=pltpu.CompilerParams(needs_layout_passes=False),
    )
    def sc_kernel(idx_hbm, val_hbm, part_hbm,
                  idx0, idx1, val0, val1, acc, si0, si1, sv0, sv1):
        cid = lax.axis_index("c")
        sid = lax.axis_index("s")
        wid = sid * NC + cid
        base = pl.multiple_of(wid * chunk, 8)

        def zero_body(i, _):
            acc[pl.ds(i * LANES, LANES)] = jnp.zeros((LANES,), jnp.float32)
            return 0

        lax.fori_loop(0, num_segments // LANES, zero_body, 0)

        idx_bufs = (idx0, idx1)
        val_bufs = (val0, val1)
        idx_sems = (si0, si1)
        val_sems = (sv0, sv1)

        def start(widx):
            p = widx % 2
            wb = pl.multiple_of(base + widx * w, 8)
            di = pltpu.async_copy(idx_hbm.at[pl.ds(wb, w)], idx_bufs[p], idx_sems[p])
            dv = pltpu.async_copy(val_hbm.at[pl.ds(wb, w)], val_bufs[p], val_sems[p])
            return di, dv

        pos0 = lax.iota(jnp.int32, LANES) * sub
        pending = start(0)

        for widx in range(nwin):
            p = widx % 2
            di, dv = pending
            di.wait()
            dv.wait()
            if widx + 1 < nwin:
                pending = start(widx + 1)
            idxb = idx_bufs[p]
            valb = val_bufs[p]

            def step(t, pos):
                iv = plsc.load_gather(idxb, [pos])
                vv = plsc.load_gather(valb, [pos])
                plsc.addupdate_scatter(acc, [iv], vv)
                return pos + 1

            lax.fori_loop(0, sub, step, pos0, unroll=8)

        pltpu.sync_copy(acc, part_hbm.at[wid])

    return sc_kernel(idx, vals)


def _tc_combine(tensor2d, partials):
    def body(t_ref, p_ref, o_ref):
        o_ref[...] = t_ref[...] - jnp.sum(p_ref[...], axis=0, keepdims=True)

    return pl.pallas_call(
        body,
        out_shape=jax.ShapeDtypeStruct(tensor2d.shape, tensor2d.dtype),
    )(tensor2d, partials)


def kernel(tensor, batch_idx, atomic_numbers, element_references):
    num_segments = tensor.shape[0]
    idx = batch_idx.astype(jnp.int32)
    vals = atomic_numbers.astype(jnp.float32)
    partials = _sc_partials(idx, vals, num_segments)
    out2d = _tc_combine(tensor.reshape(1, -1), partials)
    return out2d.reshape(tensor.shape)
